# baseline (speedup 1.0000x reference)
import functools

import jax
import jax.numpy as jnp
from jax import lax
from jax.experimental import pallas as pl
from jax.experimental.pallas import tpu as pltpu
from jax.experimental.pallas import tpu_sc as plsc

N_NODES = 10000
D = 128
N_EDGES = 320000

NC, NS = 2, 16
NW = NC * NS
CHUNK = 128
GROUP = 16
NGROUPS = 5
CHUNKS_PER_W = GROUP * NGROUPS
EDGES_PER_W = CHUNK * CHUNKS_PER_W
E_PAD = NW * EDGES_PER_W
ACC_ROWS = 10240
DUMMY_ROW = N_NODES

ZERO_ROWS_PER_SUB = ACC_ROWS // NS
OUT_ROWS_PER_SUB = ACC_ROWS // NS


def _sc_aggregate(x, src, dst):
  mesh = plsc.VectorSubcoreMesh(core_axis_name="c", subcore_axis_name="s")

  @functools.partial(
      pl.kernel,
      out_type=jax.ShapeDtypeStruct((NC, ACC_ROWS, D), jnp.float32),
      mesh=mesh,
      scratch_types=[
          pltpu.VMEM((2, GROUP, CHUNK), jnp.int32),
          pltpu.VMEM((2, GROUP, CHUNK), jnp.int32),
          pltpu.VMEM((CHUNK, D), jnp.float32),
          pltpu.VMEM((CHUNK, D), jnp.float32),
          pltpu.VMEM_SHARED((ACC_ROWS, D), jnp.float32),
          pltpu.SemaphoreType.DMA,
          pltpu.SemaphoreType.DMA,
          pltpu.SemaphoreType.DMA,
          pltpu.SemaphoreType.DMA,
      ],
  )
  def agg(x_hbm, src_hbm, dst_hbm, out_hbm, s_idx, d_idx, b0, b1, acc,
          sem0, sem1, semi, semj):
    cid = lax.axis_index("c")
    sid = lax.axis_index("s")
    wid = cid * NS + sid

    ci = pltpu.async_copy(src_hbm.at[wid, pl.ds(0, GROUP)], s_idx.at[0], semi)
    cd = pltpu.async_copy(dst_hbm.at[wid, pl.ds(0, GROUP)], d_idx.at[0], semj)

    @pl.loop(0, CHUNK)
    def _(r):
      @pl.loop(0, D, step=16)
      def _(c):
        b0[r, pl.ds(c, 16)] = jnp.zeros((16,), jnp.float32)

    @pl.loop(0, ZERO_ROWS_PER_SUB // CHUNK)
    def _(k):
      pltpu.sync_copy(
          b0, acc.at[pl.ds(sid * ZERO_ROWS_PER_SUB + k * CHUNK, CHUNK)])

    ci.wait()
    cd.wait()
    plsc.subcore_barrier()

    def start_gather(p, j, buf, sem):
      return pltpu.async_copy(x_hbm.at[s_idx.at[p, j]], buf, sem)

    def wait_gather(p, j, buf, sem):
      pltpu.make_async_copy(x_hbm.at[s_idx.at[p, j]], buf, sem).wait()

    def scatter_add(p, j, buf):
      pltpu.sync_copy(buf, acc.at[d_idx.at[p, j]], add=True)

    for g in range(NGROUPS):
      p = g & 1
      pend = None
      if g + 1 < NGROUPS:
        off = (g + 1) * GROUP
        pend = (
            pltpu.async_copy(src_hbm.at[wid, pl.ds(off, GROUP)],
                             s_idx.at[1 - p], semi),
            pltpu.async_copy(dst_hbm.at[wid, pl.ds(off, GROUP)],
                             d_idx.at[1 - p], semj),
        )

      start_gather(p, 0, b0, sem0)

      @pl.loop(0, GROUP - 2, step=2)
      def _(j, p=p):
        c1 = start_gather(p, j + 1, b1, sem1)
        wait_gather(p, j, b0, sem0)
        scatter_add(p, j, b0)
        start_gather(p, j + 2, b0, sem0)
        c1.wait()
        scatter_add(p, j + 1, b1)

      cl = start_gather(p, GROUP - 1, b1, sem1)
      wait_gather(p, GROUP - 2, b0, sem0)
      scatter_add(p, GROUP - 2, b0)
      cl.wait()
      scatter_add(p, GROUP - 1, b1)

      if pend is not None:
        pend[0].wait()
        pend[1].wait()

    plsc.subcore_barrier()

    rbase = sid * OUT_ROWS_PER_SUB
    pltpu.sync_copy(acc.at[pl.ds(rbase, OUT_ROWS_PER_SUB)],
                    out_hbm.at[cid, pl.ds(rbase, OUT_ROWS_PER_SUB)])

  return agg(x, src, dst)


def _tc_combine(partials, W):
  def body(p_ref, w_ref, o_ref):
    a = p_ref[0, :N_NODES] + p_ref[1, :N_NODES]
    o_ref[...] = lax.dot_general(
        a, w_ref[...], (((1,), (1,)), ((), ())),
        preferred_element_type=jnp.float32)

  return pl.pallas_call(
      body,
      out_shape=jax.ShapeDtypeStruct((N_NODES, D), jnp.float32),
  )(partials, W)


def kernel(x, W, edge_index, counts, out_edge_index, layer_i):
  del counts, out_edge_index, layer_i
  pad = E_PAD - N_EDGES
  src = jnp.concatenate([edge_index[0], jnp.zeros((pad,), jnp.int32)])
  dst = jnp.concatenate([edge_index[1],
                         jnp.full((pad,), DUMMY_ROW, jnp.int32)])
  src = src.reshape(NW, CHUNKS_PER_W, CHUNK)
  dst = dst.reshape(NW, CHUNKS_PER_W, CHUNK)
  partials = _sc_aggregate(x, src, dst)
  return _tc_combine(partials, W)


# speedup vs baseline: 1.1499x; 1.1499x over previous
import functools

import jax
import jax.numpy as jnp
from jax import lax
from jax.experimental import pallas as pl
from jax.experimental.pallas import tpu as pltpu
from jax.experimental.pallas import tpu_sc as plsc

N_NODES = 10000
D = 128
N_EDGES = 320000

NC, NS = 2, 16
CHUNK = 128
CHUNKS_SC0 = 93
CHUNKS_SC1 = 65
E_PAD = NS * (CHUNKS_SC0 + CHUNKS_SC1) * CHUNK
SC0_EDGES = NS * CHUNKS_SC0 * CHUNK
ACC_ROWS = 10240
DUMMY_ROW = N_NODES

ZERO_ROWS_PER_SUB = ACC_ROWS // NS
OUT_ROWS_PER_SUB = ACC_ROWS // NS


def _sc_aggregate(x, src, dst):
  mesh = plsc.VectorSubcoreMesh(core_axis_name="c", subcore_axis_name="s")

  @functools.partial(
      pl.kernel,
      out_type=jax.ShapeDtypeStruct((NC, ACC_ROWS, D), jnp.float32),
      mesh=mesh,
      scratch_types=[
          pltpu.VMEM((CHUNK,), jnp.int32),
          pltpu.VMEM((1, CHUNK), jnp.int32),
          pltpu.VMEM((CHUNK, D), jnp.float32),
          pltpu.VMEM_SHARED((ACC_ROWS, D), jnp.float32),
      ],
  )
  def agg(x_hbm, src_hbm, dst_hbm, out_hbm, s_idx, d_idx, rows, acc):
    cid = lax.axis_index("c")
    sid = lax.axis_index("s")

    @pl.loop(0, CHUNK)
    def _(r):
      @pl.loop(0, D, step=16)
      def _(c):
        rows[r, pl.ds(c, 16)] = jnp.zeros((16,), jnp.float32)

    @pl.loop(0, ZERO_ROWS_PER_SUB // CHUNK)
    def _(k):
      pltpu.sync_copy(
          rows, acc.at[pl.ds(sid * ZERO_ROWS_PER_SUB + k * CHUNK, CHUNK)])

    plsc.subcore_barrier()

    base = jnp.where(cid == 0, sid * (CHUNKS_SC0 * CHUNK),
                     SC0_EDGES + sid * (CHUNKS_SC1 * CHUNK))
    nchunks = jnp.where(cid == 0, CHUNKS_SC0, CHUNKS_SC1)

    @pl.loop(0, nchunks)
    def _(j):
      off = base + j * CHUNK
      pltpu.sync_copy(src_hbm.at[pl.ds(off, CHUNK)], s_idx)
      pltpu.sync_copy(dst_hbm.at[pl.ds(off, CHUNK)], d_idx.at[0])
      pltpu.sync_copy(x_hbm.at[s_idx], rows)
      pltpu.sync_copy(rows, acc.at[d_idx.at[0]], add=True)

    plsc.subcore_barrier()

    rbase = sid * OUT_ROWS_PER_SUB
    pltpu.sync_copy(acc.at[pl.ds(rbase, OUT_ROWS_PER_SUB)],
                    out_hbm.at[cid, pl.ds(rbase, OUT_ROWS_PER_SUB)])

  return agg(x, src, dst)


def _tc_combine(partials, W):
  def body(p_ref, w_ref, o_ref):
    a = p_ref[0, :N_NODES] + p_ref[1, :N_NODES]
    o_ref[...] = lax.dot_general(
        a, w_ref[...], (((1,), (1,)), ((), ())),
        preferred_element_type=jnp.float32)

  return pl.pallas_call(
      body,
      out_shape=jax.ShapeDtypeStruct((N_NODES, D), jnp.float32),
  )(partials, W)


def kernel(x, W, edge_index, counts, out_edge_index, layer_i):
  del counts, out_edge_index, layer_i
  pad = E_PAD - N_EDGES
  src = jnp.concatenate([edge_index[0], jnp.zeros((pad,), jnp.int32)])
  dst = jnp.concatenate([edge_index[1],
                         jnp.full((pad,), DUMMY_ROW, jnp.int32)])
  partials = _sc_aggregate(x, src, dst)
  return _tc_combine(partials, W)


# speedup vs baseline: 1.3284x; 1.1552x over previous
import functools

import jax
import jax.numpy as jnp
from jax import lax
from jax.experimental import pallas as pl
from jax.experimental.pallas import tpu as pltpu
from jax.experimental.pallas import tpu_sc as plsc

N_NODES = 10000
D = 128
N_EDGES = 320000

NC, NS = 2, 16
CHUNK = 128
N_CHUNKS = N_EDGES // CHUNK
CHUNKS_SC0 = 96
SC0_CHUNKS = NS * CHUNKS_SC0
SC1_CHUNKS = N_CHUNKS - SC0_CHUNKS
ACC_ROWS = 10240

ZERO_ROWS_PER_SUB = ACC_ROWS // NS
OUT_ROWS_PER_SUB = ACC_ROWS // NS


def _sc_aggregate(x, edges):
  mesh = plsc.VectorSubcoreMesh(core_axis_name="c", subcore_axis_name="s")

  @functools.partial(
      pl.kernel,
      out_type=jax.ShapeDtypeStruct((NC, ACC_ROWS, D), jnp.float32),
      mesh=mesh,
      scratch_types=[
          pltpu.VMEM((CHUNK,), jnp.int32),
          pltpu.VMEM((1, CHUNK), jnp.int32),
          pltpu.VMEM((CHUNK, D), jnp.float32),
          pltpu.VMEM_SHARED((ACC_ROWS, D), jnp.float32),
      ],
  )
  def agg(x_hbm, edges_hbm, out_hbm, s_idx, d_idx, rows, acc):
    cid = lax.axis_index("c")
    sid = lax.axis_index("s")

    @pl.loop(0, CHUNK)
    def _(r):
      @pl.loop(0, D, step=16)
      def _(c):
        rows[r, pl.ds(c, 16)] = jnp.zeros((16,), jnp.float32)

    @pl.loop(0, ZERO_ROWS_PER_SUB // CHUNK)
    def _(k):
      pltpu.sync_copy(
          rows, acc.at[pl.ds(sid * ZERO_ROWS_PER_SUB + k * CHUNK, CHUNK)])

    plsc.subcore_barrier()

    base_chunk = jnp.where(
        cid == 0, sid * CHUNKS_SC0,
        SC0_CHUNKS + sid * 60 + jnp.minimum(sid, 4))
    nchunks = jnp.where(cid == 0, CHUNKS_SC0,
                        jnp.where(sid < 4, 61, 60))
    base = base_chunk * CHUNK

    @pl.loop(0, nchunks)
    def _(j):
      off = base + j * CHUNK
      pltpu.sync_copy(edges_hbm.at[pl.ds(off, CHUNK)], s_idx)
      pltpu.sync_copy(edges_hbm.at[pl.ds(N_EDGES + off, CHUNK)], d_idx.at[0])
      pltpu.sync_copy(x_hbm.at[s_idx], rows)
      pltpu.sync_copy(rows, acc.at[d_idx.at[0]], add=True)

    plsc.subcore_barrier()

    rbase = sid * OUT_ROWS_PER_SUB
    pltpu.sync_copy(acc.at[pl.ds(rbase, OUT_ROWS_PER_SUB)],
                    out_hbm.at[cid, pl.ds(rbase, OUT_ROWS_PER_SUB)])

  return agg(x, edges)


def _tc_combine(partials, W):
  def body(p_ref, w_ref, o_ref):
    a = p_ref[0, :N_NODES] + p_ref[1, :N_NODES]
    o_ref[...] = lax.dot_general(
        a, w_ref[...], (((1,), (1,)), ((), ())),
        preferred_element_type=jnp.float32)

  return pl.pallas_call(
      body,
      out_shape=jax.ShapeDtypeStruct((N_NODES, D), jnp.float32),
  )(partials, W)


def kernel(x, W, edge_index, counts, out_edge_index, layer_i):
  del counts, out_edge_index, layer_i
  partials = _sc_aggregate(x, edge_index.reshape(-1))
  return _tc_combine(partials, W)


# speedup vs baseline: 1.5631x; 1.1767x over previous
import functools

import jax
import jax.numpy as jnp
from jax import lax
from jax.experimental import pallas as pl
from jax.experimental.pallas import tpu as pltpu
from jax.experimental.pallas import tpu_sc as plsc

N_NODES = 10000
D = 128
N_EDGES = 320000

NC, NS = 2, 16
CHUNK = 128
N_CHUNKS = N_EDGES // CHUNK
CHUNKS_SC0 = 79
SC0_CHUNKS = NS * CHUNKS_SC0
SC1_CHUNKS = N_CHUNKS - SC0_CHUNKS
ACC_ROWS = 10240

ZERO_ROWS_PER_SUB = ACC_ROWS // NS
OUT_ROWS_PER_SUB = ACC_ROWS // NS


def _sc_aggregate(x, edges):
  mesh = plsc.VectorSubcoreMesh(core_axis_name="c", subcore_axis_name="s")

  @functools.partial(
      pl.kernel,
      out_type=jax.ShapeDtypeStruct((NC, ACC_ROWS, D), jnp.float32),
      mesh=mesh,
      scratch_types=[
          pltpu.VMEM((CHUNK,), jnp.int32),
          pltpu.VMEM((1, CHUNK), jnp.int32),
          pltpu.VMEM((CHUNK, D), jnp.float32),
          pltpu.VMEM_SHARED((ACC_ROWS, D), jnp.float32),
      ],
  )
  def agg(x_hbm, edges_hbm, out_hbm, s_idx, d_idx, rows, acc):
    cid = lax.axis_index("c")
    sid = lax.axis_index("s")

    @pl.loop(0, CHUNK)
    def _(r):
      @pl.loop(0, D, step=16)
      def _(c):
        rows[r, pl.ds(c, 16)] = jnp.zeros((16,), jnp.float32)

    @pl.loop(0, ZERO_ROWS_PER_SUB // CHUNK)
    def _(k):
      pltpu.sync_copy(
          rows, acc.at[pl.ds(sid * ZERO_ROWS_PER_SUB + k * CHUNK, CHUNK)])

    plsc.subcore_barrier()

    base_chunk = jnp.where(
        cid == 0, sid * CHUNKS_SC0,
        SC0_CHUNKS + sid * 77 + jnp.minimum(sid, 4))
    nchunks = jnp.where(cid == 0, CHUNKS_SC0,
                        jnp.where(sid < 4, 78, 77))
    base = base_chunk * CHUNK

    @pl.loop(0, nchunks)
    def _(j):
      off = base + j * CHUNK
      pltpu.sync_copy(edges_hbm.at[pl.ds(off, CHUNK)], s_idx)
      pltpu.sync_copy(edges_hbm.at[pl.ds(N_EDGES + off, CHUNK)], d_idx.at[0])
      pltpu.sync_copy(x_hbm.at[s_idx], rows)
      pltpu.sync_copy(rows, acc.at[d_idx.at[0]], add=True)

    plsc.subcore_barrier()

    rbase = sid * OUT_ROWS_PER_SUB
    pltpu.sync_copy(acc.at[pl.ds(rbase, OUT_ROWS_PER_SUB)],
                    out_hbm.at[cid, pl.ds(rbase, OUT_ROWS_PER_SUB)])

  return agg(x, edges)


def _tc_combine(partials, W):
  def body(p_ref, w_ref, o_ref):
    a = p_ref[0, :N_NODES] + p_ref[1, :N_NODES]
    o_ref[...] = lax.dot_general(
        a, w_ref[...], (((1,), (1,)), ((), ())),
        preferred_element_type=jnp.float32)

  return pl.pallas_call(
      body,
      out_shape=jax.ShapeDtypeStruct((N_NODES, D), jnp.float32),
  )(partials, W)


def kernel(x, W, edge_index, counts, out_edge_index, layer_i):
  del counts, out_edge_index, layer_i
  partials = _sc_aggregate(x, edge_index.reshape(-1))
  return _tc_combine(partials, W)


# speedup vs baseline: 2.1186x; 1.3554x over previous
import functools

import jax
import jax.numpy as jnp
from jax import lax
from jax.experimental import pallas as pl
from jax.experimental.pallas import tpu as pltpu
from jax.experimental.pallas import tpu_sc as plsc

N_NODES = 10000
D = 128
N_EDGES = 320000

NC, NS = 2, 16
CHUNK = 128
N_CHUNKS = N_EDGES // CHUNK
CHUNKS_SC0 = 79
SC0_CHUNKS = NS * CHUNKS_SC0
SC1_CHUNKS = N_CHUNKS - SC0_CHUNKS
ACC_ROWS = 10240

ZERO_ROWS_PER_SUB = ACC_ROWS // NS
OUT_ROWS_PER_SUB = ACC_ROWS // NS


def _sc_aggregate(x, edges):
  mesh = plsc.VectorSubcoreMesh(core_axis_name="c", subcore_axis_name="s")

  @functools.partial(
      pl.kernel,
      out_type=jax.ShapeDtypeStruct((NC, ACC_ROWS, D), jnp.float32),
      mesh=mesh,
      scratch_types=[
          pltpu.VMEM((2, CHUNK), jnp.int32),
          pltpu.VMEM((2, CHUNK), jnp.int32),
          pltpu.VMEM((CHUNK, D), jnp.float32),
          pltpu.VMEM_SHARED((ACC_ROWS, D), jnp.float32),
          pltpu.SemaphoreType.DMA,
          pltpu.SemaphoreType.DMA,
      ],
  )
  def agg(x_hbm, edges_hbm, out_hbm, s_idx, d_idx, rows, acc, sem_s, sem_d):
    cid = lax.axis_index("c")
    sid = lax.axis_index("s")

    @pl.loop(0, CHUNK)
    def _(r):
      @pl.loop(0, D, step=16)
      def _(c):
        rows[r, pl.ds(c, 16)] = jnp.zeros((16,), jnp.float32)

    @pl.loop(0, ZERO_ROWS_PER_SUB // CHUNK)
    def _(k):
      pltpu.sync_copy(
          rows, acc.at[pl.ds(sid * ZERO_ROWS_PER_SUB + k * CHUNK, CHUNK)])

    plsc.subcore_barrier()

    base_chunk = jnp.where(
        cid == 0, sid * CHUNKS_SC0,
        SC0_CHUNKS + sid * 77 + jnp.minimum(sid, 4))
    nchunks = jnp.where(cid == 0, CHUNKS_SC0,
                        jnp.where(sid < 4, 78, 77))
    base = base_chunk * CHUNK

    def idx_start(j, p):
      off = base + j * CHUNK
      pltpu.async_copy(edges_hbm.at[pl.ds(off, CHUNK)], s_idx.at[p], sem_s)
      pltpu.async_copy(edges_hbm.at[pl.ds(N_EDGES + off, CHUNK)],
                       d_idx.at[p], sem_d)

    def idx_wait(j, p):
      off = base + j * CHUNK
      pltpu.make_async_copy(edges_hbm.at[pl.ds(off, CHUNK)], s_idx.at[p],
                            sem_s).wait()
      pltpu.make_async_copy(edges_hbm.at[pl.ds(N_EDGES + off, CHUNK)],
                            d_idx.at[p], sem_d).wait()

    idx_start(0, 0)

    @pl.loop(0, nchunks)
    def _(j):
      p = jnp.bitwise_and(j, 1)
      idx_wait(j, p)

      @pl.when(j + 1 < nchunks)
      def _():
        idx_start(j + 1, 1 - p)

      pltpu.sync_copy(x_hbm.at[s_idx.at[p]], rows)
      pltpu.sync_copy(rows, acc.at[d_idx.at[p]], add=True)

    plsc.subcore_barrier()

    rbase = sid * OUT_ROWS_PER_SUB
    pltpu.sync_copy(acc.at[pl.ds(rbase, OUT_ROWS_PER_SUB)],
                    out_hbm.at[cid, pl.ds(rbase, OUT_ROWS_PER_SUB)])

  return agg(x, edges)


def _tc_combine(partials, W):
  def body(p_ref, w_ref, o_ref):
    a = p_ref[0, :N_NODES] + p_ref[1, :N_NODES]
    o_ref[...] = lax.dot_general(
        a, w_ref[...], (((1,), (1,)), ((), ())),
        preferred_element_type=jnp.float32)

  return pl.pallas_call(
      body,
      out_shape=jax.ShapeDtypeStruct((N_NODES, D), jnp.float32),
  )(partials, W)


def kernel(x, W, edge_index, counts, out_edge_index, layer_i):
  del counts, out_edge_index, layer_i
  partials = _sc_aggregate(x, edge_index.reshape(-1))
  return _tc_combine(partials, W)


# speedup vs baseline: 2.7528x; 1.2993x over previous
import functools

import jax
import jax.numpy as jnp
from jax import lax
from jax.experimental import pallas as pl
from jax.experimental.pallas import tpu as pltpu
from jax.experimental.pallas import tpu_sc as plsc

N_NODES = 10000
D = 128
N_EDGES = 320000

NC, NS = 2, 16
CHUNK = 128
N_CHUNKS = N_EDGES // CHUNK
CHUNKS_SC0 = 79
SC0_CHUNKS = NS * CHUNKS_SC0
SC1_CHUNKS = N_CHUNKS - SC0_CHUNKS
ACC_ROWS = 10240

ZERO_ROWS_PER_SUB = ACC_ROWS // NS
OUT_ROWS_PER_SUB = ACC_ROWS // NS


def _sc_aggregate(x, edges):
  mesh = plsc.VectorSubcoreMesh(core_axis_name="c", subcore_axis_name="s")

  @functools.partial(
      pl.kernel,
      out_type=jax.ShapeDtypeStruct((NC, ACC_ROWS, D), jnp.float32),
      mesh=mesh,
      scratch_types=[
          pltpu.VMEM((2, CHUNK), jnp.int32),
          pltpu.VMEM((2, CHUNK), jnp.int32),
          pltpu.VMEM((2, CHUNK, D), jnp.float32),
          pltpu.VMEM_SHARED((ACC_ROWS, D), jnp.float32),
          pltpu.SemaphoreType.DMA,
          pltpu.SemaphoreType.DMA,
          pltpu.SemaphoreType.DMA,
      ],
  )
  def agg(x_hbm, edges_hbm, out_hbm, s_idx, d_idx, rows, acc,
          sem_s, sem_d, sem_g):
    cid = lax.axis_index("c")
    sid = lax.axis_index("s")

    @pl.loop(0, CHUNK)
    def _(r):
      @pl.loop(0, D, step=16)
      def _(c):
        rows[0, r, pl.ds(c, 16)] = jnp.zeros((16,), jnp.float32)

    @pl.loop(0, ZERO_ROWS_PER_SUB // CHUNK)
    def _(k):
      pltpu.sync_copy(
          rows.at[0],
          acc.at[pl.ds(sid * ZERO_ROWS_PER_SUB + k * CHUNK, CHUNK)])

    plsc.subcore_barrier()

    base_chunk = jnp.where(
        cid == 0, sid * CHUNKS_SC0,
        SC0_CHUNKS + sid * 77 + jnp.minimum(sid, 4))
    nchunks = jnp.where(cid == 0, CHUNKS_SC0,
                        jnp.where(sid < 4, 78, 77))
    base = base_chunk * CHUNK

    def idx_start(j, p):
      off = base + j * CHUNK
      pltpu.async_copy(edges_hbm.at[pl.ds(off, CHUNK)], s_idx.at[p], sem_s)
      pltpu.async_copy(edges_hbm.at[pl.ds(N_EDGES + off, CHUNK)],
                       d_idx.at[p], sem_d)

    def idx_wait(j, p):
      off = base + j * CHUNK
      pltpu.make_async_copy(edges_hbm.at[pl.ds(off, CHUNK)], s_idx.at[p],
                            sem_s).wait()
      pltpu.make_async_copy(edges_hbm.at[pl.ds(N_EDGES + off, CHUNK)],
                            d_idx.at[p], sem_d).wait()

    def gather_start(p):
      pltpu.async_copy(x_hbm.at[s_idx.at[p]], rows.at[p], sem_g)

    def gather_wait(p):
      pltpu.make_async_copy(x_hbm.at[s_idx.at[p]], rows.at[p], sem_g).wait()

    idx_start(0, 0)
    idx_wait(0, 0)
    idx_start(1, 1)
    gather_start(0)

    @pl.loop(0, nchunks)
    def _(j):
      p = jnp.bitwise_and(j, 1)
      gather_wait(p)

      @pl.when(j + 1 < nchunks)
      def _():
        idx_wait(j + 1, 1 - p)
        gather_start(1 - p)

      pltpu.sync_copy(rows.at[p], acc.at[d_idx.at[p]], add=True)

      @pl.when(j + 2 < nchunks)
      def _():
        idx_start(j + 2, p)

    plsc.subcore_barrier()

    rbase = sid * OUT_ROWS_PER_SUB
    pltpu.sync_copy(acc.at[pl.ds(rbase, OUT_ROWS_PER_SUB)],
                    out_hbm.at[cid, pl.ds(rbase, OUT_ROWS_PER_SUB)])

  return agg(x, edges)


def _tc_combine(partials, W):
  def body(p_ref, w_ref, o_ref):
    a = p_ref[0, :N_NODES] + p_ref[1, :N_NODES]
    o_ref[...] = lax.dot_general(
        a, w_ref[...], (((1,), (1,)), ((), ())),
        preferred_element_type=jnp.float32)

  return pl.pallas_call(
      body,
      out_shape=jax.ShapeDtypeStruct((N_NODES, D), jnp.float32),
  )(partials, W)


def kernel(x, W, edge_index, counts, out_edge_index, layer_i):
  del counts, out_edge_index, layer_i
  partials = _sc_aggregate(x, edge_index.reshape(-1))
  return _tc_combine(partials, W)


# speedup vs baseline: 3.0001x; 1.0899x over previous
import functools

import jax
import jax.numpy as jnp
from jax import lax
from jax.experimental import pallas as pl
from jax.experimental.pallas import tpu as pltpu
from jax.experimental.pallas import tpu_sc as plsc

N_NODES = 10000
D = 128
N_EDGES = 320000

NC, NS = 2, 16
CHUNK = 128
N_CHUNKS = N_EDGES // CHUNK
CHUNKS_SC0 = 79
SC0_CHUNKS = NS * CHUNKS_SC0
SC1_CHUNKS = N_CHUNKS - SC0_CHUNKS
ACC_ROWS = 10112

ROWS_PER_SUB = ACC_ROWS // NS


def _sc_aggregate(x, edges):
  mesh = plsc.VectorSubcoreMesh(core_axis_name="c", subcore_axis_name="s")

  @functools.partial(
      pl.kernel,
      out_type=jax.ShapeDtypeStruct((NC, ACC_ROWS, D), jnp.float32),
      mesh=mesh,
      scratch_types=[
          pltpu.VMEM((3, CHUNK), jnp.int32),
          pltpu.VMEM((3, CHUNK), jnp.int32),
          pltpu.VMEM((3, CHUNK, D), jnp.float32),
          pltpu.VMEM_SHARED((ACC_ROWS, D), jnp.float32),
          pltpu.SemaphoreType.DMA,
          pltpu.SemaphoreType.DMA,
          pltpu.SemaphoreType.DMA,
      ],
  )
  def agg(x_hbm, edges_hbm, out_hbm, s_idx, d_idx, rows, acc,
          sem_s, sem_d, sem_g):
    cid = lax.axis_index("c")
    sid = lax.axis_index("s")

    @pl.loop(0, CHUNK)
    def _(r):
      @pl.loop(0, D, step=16)
      def _(c):
        rows[0, r, pl.ds(c, 16)] = jnp.zeros((16,), jnp.float32)

    @pl.loop(0, ROWS_PER_SUB // CHUNK)
    def _(k):
      pltpu.sync_copy(
          rows.at[0],
          acc.at[pl.ds(sid * ROWS_PER_SUB + k * CHUNK, CHUNK)])

    pltpu.sync_copy(
        rows.at[0, pl.ds(0, ROWS_PER_SUB % CHUNK)],
        acc.at[pl.ds(sid * ROWS_PER_SUB + (ROWS_PER_SUB // CHUNK) * CHUNK,
                     ROWS_PER_SUB % CHUNK)])

    plsc.subcore_barrier()

    base_chunk = jnp.where(
        cid == 0, sid * CHUNKS_SC0,
        SC0_CHUNKS + sid * 77 + jnp.minimum(sid, 4))
    nchunks = jnp.where(cid == 0, CHUNKS_SC0,
                        jnp.where(sid < 4, 78, 77))
    base = base_chunk * CHUNK

    def idx_start(j, p):
      off = base + j * CHUNK
      pltpu.async_copy(edges_hbm.at[pl.ds(off, CHUNK)], s_idx.at[p], sem_s)
      pltpu.async_copy(edges_hbm.at[pl.ds(N_EDGES + off, CHUNK)],
                       d_idx.at[p], sem_d)

    def idx_wait(j, p):
      off = base + j * CHUNK
      pltpu.make_async_copy(edges_hbm.at[pl.ds(off, CHUNK)], s_idx.at[p],
                            sem_s).wait()
      pltpu.make_async_copy(edges_hbm.at[pl.ds(N_EDGES + off, CHUNK)],
                            d_idx.at[p], sem_d).wait()

    def gather_start(p):
      pltpu.async_copy(x_hbm.at[s_idx.at[p]], rows.at[p], sem_g)

    def gather_wait(p):
      pltpu.make_async_copy(x_hbm.at[s_idx.at[p]], rows.at[p], sem_g).wait()

    idx_start(0, 0)
    idx_start(1, 1)
    idx_start(2, 2)
    idx_wait(0, 0)
    gather_start(0)
    idx_wait(1, 1)
    gather_start(1)

    @pl.loop(0, nchunks)
    def _(j):
      p = lax.rem(j, 3)
      gather_wait(p)

      @pl.when(j + 2 < nchunks)
      def _():
        q = lax.rem(j + 2, 3)
        idx_wait(j + 2, q)
        gather_start(q)

      pltpu.sync_copy(rows.at[p], acc.at[d_idx.at[p]], add=True)

      @pl.when(j + 3 < nchunks)
      def _():
        idx_start(j + 3, p)

    plsc.subcore_barrier()

    rbase = sid * ROWS_PER_SUB
    pltpu.sync_copy(acc.at[pl.ds(rbase, ROWS_PER_SUB)],
                    out_hbm.at[cid, pl.ds(rbase, ROWS_PER_SUB)])

  return agg(x, edges)


def _tc_combine(partials, W):
  def body(p_ref, w_ref, o_ref):
    a = p_ref[0, :N_NODES] + p_ref[1, :N_NODES]
    o_ref[...] = lax.dot_general(
        a, w_ref[...], (((1,), (1,)), ((), ())),
        preferred_element_type=jnp.float32)

  return pl.pallas_call(
      body,
      out_shape=jax.ShapeDtypeStruct((N_NODES, D), jnp.float32),
  )(partials, W)


def kernel(x, W, edge_index, counts, out_edge_index, layer_i):
  del counts, out_edge_index, layer_i
  partials = _sc_aggregate(x, edge_index.reshape(-1))
  return _tc_combine(partials, W)
